# baseline (speedup 1.0000x reference)
import jax
import jax.numpy as jnp
from jax import lax
from jax.experimental import pallas as pl
from jax.experimental.pallas import tpu as pltpu

_TILE_N = 2048


def _matvec_kernel(wt_ref, b_ref, h_ref, o_ref):
    acc = lax.dot_general(
        h_ref[...], wt_ref[...],
        dimension_numbers=(((1,), (0,)), ((), ())),
        preferred_element_type=jnp.float32,
    )
    o_ref[...] = (acc + b_ref[0, 0]).astype(o_ref.dtype)


def kernel(h, weight, bias):
    n, d = h.shape
    tile_n = _TILE_N if n % _TILE_N == 0 else (n if n <= 1024 else 1024)
    num_tiles = pl.cdiv(n, tile_n)
    wt = weight.reshape(d, 1).astype(jnp.float32)
    b2 = bias.reshape(1, 1).astype(jnp.float32)

    bytes_accessed = n * d * h.dtype.itemsize + d * 4 + n * h.dtype.itemsize
    cost = pl.CostEstimate(flops=2 * n * d, transcendentals=0,
                           bytes_accessed=bytes_accessed)

    return pl.pallas_call(
        _matvec_kernel,
        out_shape=jax.ShapeDtypeStruct((n, 1), h.dtype),
        grid=(num_tiles,),
        in_specs=[
            pl.BlockSpec((d, 1), lambda i: (0, 0)),
            pl.BlockSpec(memory_space=pltpu.SMEM),
            pl.BlockSpec((tile_n, d), lambda i: (i, 0)),
        ],
        out_specs=pl.BlockSpec((tile_n, 1), lambda i: (i, 0)),
        compiler_params=pltpu.CompilerParams(
            dimension_semantics=("parallel",),
        ),
        cost_estimate=cost,
    )(wt, b2, h)


# speedup vs baseline: 1.3332x; 1.3332x over previous
import jax
import jax.numpy as jnp
from jax import lax
from jax.experimental import pallas as pl
from jax.experimental.pallas import tpu as pltpu

_TILE_N = 1024


def _matvec2_kernel(w_ref, b_ref, ha_ref, hb_ref, o_ref):
    w = w_ref[...]
    b = b_ref[0, 0]
    acc_a = lax.dot_general(
        w, ha_ref[...],
        dimension_numbers=(((1,), (1,)), ((), ())),
        preferred_element_type=jnp.float32,
    )
    acc_b = lax.dot_general(
        w, hb_ref[...],
        dimension_numbers=(((1,), (1,)), ((), ())),
        preferred_element_type=jnp.float32,
    )
    o_ref[...] = (jnp.concatenate([acc_a, acc_b], axis=0) + b).astype(o_ref.dtype)


def kernel(h, weight, bias):
    n, d = h.shape
    b2 = bias.reshape(1, 1).astype(jnp.float32)
    w = weight.astype(jnp.float32)

    tile_n = _TILE_N
    if n % (2 * tile_n) != 0:
        tile_n1 = n if n <= 1024 else 1024
        num_tiles = pl.cdiv(n, tile_n1)
        out_row = pl.pallas_call(
            lambda w_ref, b_ref, h_ref, o_ref: o_ref.__setitem__(
                ...,
                (lax.dot_general(w_ref[...], h_ref[...],
                                 dimension_numbers=(((1,), (1,)), ((), ())),
                                 preferred_element_type=jnp.float32)
                 + b_ref[0, 0]).astype(o_ref.dtype)),
            out_shape=jax.ShapeDtypeStruct((1, num_tiles * tile_n1), h.dtype),
            grid=(num_tiles,),
            in_specs=[
                pl.BlockSpec((1, d), lambda i: (0, 0)),
                pl.BlockSpec(memory_space=pltpu.SMEM),
                pl.BlockSpec((tile_n1, d), lambda i: (i, 0)),
            ],
            out_specs=pl.BlockSpec((1, tile_n1), lambda i: (0, i)),
            compiler_params=pltpu.CompilerParams(
                dimension_semantics=("parallel",)),
        )(w, b2, h)
        return out_row[0, :n].reshape(n, 1)

    num_steps = n // (2 * tile_n)
    half = num_steps

    bytes_accessed = n * d * h.dtype.itemsize + d * 4 + n * h.dtype.itemsize
    cost = pl.CostEstimate(flops=2 * n * d, transcendentals=0,
                           bytes_accessed=bytes_accessed)

    out2 = pl.pallas_call(
        _matvec2_kernel,
        out_shape=jax.ShapeDtypeStruct((2, n // 2), h.dtype),
        grid=(num_steps,),
        in_specs=[
            pl.BlockSpec((1, d), lambda i: (0, 0)),
            pl.BlockSpec(memory_space=pltpu.SMEM),
            pl.BlockSpec((tile_n, d), lambda i: (i, 0)),
            pl.BlockSpec((tile_n, d), lambda i: (half + i, 0)),
        ],
        out_specs=pl.BlockSpec((2, tile_n), lambda i: (0, i)),
        compiler_params=pltpu.CompilerParams(
            dimension_semantics=("parallel",),
        ),
        cost_estimate=cost,
    )(w, b2, h, h)

    return out2.reshape(n, 1)
